# Initial kernel scaffold; baseline (speedup 1.0000x reference)
import jax
import jax.numpy as jnp
from jax.experimental import pallas as pl


def kernel(des, tweet, num_prop, cat_prop, edge_index, edge_type, Wd, bd, Wt, bt, Wi, bi, Wrel, Wroot, brgcn, Wo1, bo1, Wo2, bo2):
    raise NotImplementedError("write your pallas kernel here")


# speedup vs baseline: 9.9370x; 9.9370x over previous
import jax
import jax.numpy as jnp
from jax import lax
from jax.experimental import pallas as pl
from jax.experimental.pallas import tpu as pltpu
from jax.experimental.pallas import tpu_sc as plsc

N = 10000
E = 320000
EMB = 128
HALF = 64
NUM_REL = 2

NTILE = 16
CHUNK = 32
NBUF = 3
ACC = 20480
ROWS_PER_TILE = ACC // NTILE
EP = 321024
PER_TILE_E = EP // NTILE
NCHUNK = PER_TILE_E // CHUNK
NGRP = NCHUNK // NBUF
NPAD = 10112
TROWS = NPAD // NTILE

_f32 = jnp.float32


def _leaky(x):
    return jnp.where(x >= 0, x, 0.01 * x)


def _make_sc_agg(with_cnt: bool):
    mesh = plsc.VectorSubcoreMesh(
        core_axis_name="c", subcore_axis_name="s", num_cores=2, num_subcores=NTILE
    )
    out_type = [jax.ShapeDtypeStruct((2 * ACC, HALF), _f32)]
    if with_cnt:
        out_type.append(jax.ShapeDtypeStruct((ACC,), _f32))

    def body(*refs):
        (xh, eidx, z2d, zc, ones_h) = refs[:5]
        nout = 2 if with_cnt else 1
        outs = refs[5:5 + nout]
        agg = outs[0]
        cnt = outs[1] if with_cnt else None
        rest = list(refs[5 + nout:])
        eidx_v = rest[0:NBUF]
        gidx_v = rest[NBUF:2 * NBUF]
        sidx_v = rest[2 * NBUF:3 * NBUF]
        rows_v = rest[3 * NBUF:4 * NBUF]
        (ones_v, acc, acc_cnt, xtab, sem_i, sem_g, sem_s, sem_c) = rest[4 * NBUF:]

        c = lax.axis_index("c")
        s = lax.axis_index("s")
        zbase = s * ROWS_PER_TILE
        tbase = s * TROWS

        pltpu.async_copy(xh.at[pl.ds(c * NPAD + tbase, TROWS)],
                         xtab.at[pl.ds(tbase, TROWS)], sem_i)
        pltpu.async_copy(z2d, acc.at[pl.ds(zbase, ROWS_PER_TILE)], sem_i)
        if with_cnt:
            @pl.when(c == 0)
            def _():
                pltpu.async_copy(zc, acc_cnt.at[pl.ds(zbase, ROWS_PER_TILE)], sem_i)
                pltpu.sync_copy(ones_h, ones_v)
        pltpu.make_async_copy(xh.at[pl.ds(c * NPAD + tbase, TROWS)],
                              xtab.at[pl.ds(tbase, TROWS)], sem_i).wait()
        pltpu.make_async_copy(z2d, acc.at[pl.ds(zbase, ROWS_PER_TILE)], sem_i).wait()
        if with_cnt:
            @pl.when(c == 0)
            def _():
                pltpu.make_async_copy(
                    zc, acc_cnt.at[pl.ds(zbase, ROWS_PER_TILE)], sem_i).wait()
        plsc.subcore_barrier()

        cbase = s * NCHUNK

        for b in range(NBUF):
            pltpu.async_copy(eidx.at[cbase + b], eidx_v[b], sem_i)

        def grp(g, carry):
            k0 = g * NBUF
            for b in range(NBUF):
                cid = cbase + k0 + b

                @pl.when(g > 0)
                def _(b=b, cid=cid):
                    pltpu.make_async_copy(rows_v[b], acc.at[sidx_v[b]], sem_s).wait()
                    if with_cnt:
                        @pl.when(c == 0)
                        def _():
                            pltpu.make_async_copy(
                                ones_v, acc_cnt.at[sidx_v[b]], sem_c).wait()

                pltpu.make_async_copy(eidx.at[cid], eidx_v[b], sem_i).wait()
                for j in range(CHUNK // 16):
                    sl = pl.ds(j * 16, 16)
                    gidx_v[b][sl] = eidx_v[b][0, sl]
                    sidx_v[b][sl] = eidx_v[b][1, sl]

                @pl.when(g < NGRP - 1)
                def _(b=b, cid=cid):
                    pltpu.async_copy(eidx.at[cid + NBUF], eidx_v[b], sem_i)

                pltpu.async_copy(xtab.at[gidx_v[b]], rows_v[b], sem_g)

            for b in range(NBUF):
                pltpu.make_async_copy(xtab.at[gidx_v[b]], rows_v[b], sem_g).wait()
                pltpu.async_copy(rows_v[b], acc.at[sidx_v[b]], sem_s, add=True)
                if with_cnt:
                    @pl.when(c == 0)
                    def _(b=b):
                        pltpu.async_copy(ones_v, acc_cnt.at[sidx_v[b]], sem_c, add=True)
            return carry

        lax.fori_loop(0, NGRP, grp, 0)

        for b in range(NBUF):
            pltpu.make_async_copy(rows_v[b], acc.at[sidx_v[b]], sem_s).wait()
            if with_cnt:
                @pl.when(c == 0)
                def _(b=b):
                    pltpu.make_async_copy(ones_v, acc_cnt.at[sidx_v[b]], sem_c).wait()
        plsc.subcore_barrier()

        pltpu.sync_copy(acc.at[pl.ds(zbase, ROWS_PER_TILE)],
                        agg.at[pl.ds(c * ACC + zbase, ROWS_PER_TILE)])
        if with_cnt:
            @pl.when(c == 0)
            def _():
                pltpu.sync_copy(acc_cnt.at[pl.ds(zbase, ROWS_PER_TILE)],
                                cnt.at[pl.ds(zbase, ROWS_PER_TILE)])

    return pl.kernel(
        body,
        out_type=out_type,
        mesh=mesh,
        scratch_types=(
            [pltpu.VMEM((2, CHUNK), jnp.int32) for _ in range(NBUF)]
            + [pltpu.VMEM((CHUNK,), jnp.int32) for _ in range(NBUF)]
            + [pltpu.VMEM((CHUNK,), jnp.int32) for _ in range(NBUF)]
            + [pltpu.VMEM((CHUNK, HALF), _f32) for _ in range(NBUF)]
            + [
                pltpu.VMEM((CHUNK,), _f32),
                pltpu.VMEM_SHARED((ACC, HALF), _f32),
                pltpu.VMEM_SHARED((ACC,), _f32),
                pltpu.VMEM_SHARED((NPAD, HALF), _f32),
                pltpu.SemaphoreType.DMA,
                pltpu.SemaphoreType.DMA,
                pltpu.SemaphoreType.DMA,
                pltpu.SemaphoreType.DMA,
            ]
        ),
        compiler_params=pltpu.CompilerParams(use_tc_tiling_on_sc=False),
    )


_TB = 1000


def _tc_input(des, tweet, Wd, bd, Wt, bt, Wi, bi):
    def body(des_r, tw_r, Wd_r, bd_r, Wt_r, bt_r, Wi_r, bi_r, out_r):
        d = _leaky(jnp.dot(des_r[...], Wd_r[...], preferred_element_type=_f32) + bd_r[...])
        t = _leaky(jnp.dot(tw_r[...], Wt_r[...], preferred_element_type=_f32) + bt_r[...])
        xc = jnp.concatenate([d, t], axis=1)
        out_r[...] = _leaky(jnp.dot(xc, Wi_r[...], preferred_element_type=_f32) + bi_r[...])

    return pl.pallas_call(
        body,
        grid=(N // _TB,),
        in_specs=[
            pl.BlockSpec((_TB, 768), lambda i: (i, 0)),
            pl.BlockSpec((_TB, 768), lambda i: (i, 0)),
            pl.BlockSpec((768, HALF), lambda i: (0, 0)),
            pl.BlockSpec((1, HALF), lambda i: (0, 0)),
            pl.BlockSpec((768, HALF), lambda i: (0, 0)),
            pl.BlockSpec((1, HALF), lambda i: (0, 0)),
            pl.BlockSpec((EMB, EMB), lambda i: (0, 0)),
            pl.BlockSpec((1, EMB), lambda i: (0, 0)),
        ],
        out_specs=pl.BlockSpec((_TB, EMB), lambda i: (i, 0)),
        out_shape=jax.ShapeDtypeStruct((N, EMB), _f32),
    )(des, tweet, Wd, bd.reshape(1, HALF), Wt, bt.reshape(1, HALF), Wi, bi.reshape(1, EMB))


def _tc_combine(x, p00, p10, p01, p11, c0, c1, Wroot, b, W0, W1, head):
    def body(*refs):
        (x_r, p00_r, p10_r, p01_r, p11_r, c0_r, c1_r, Wroot_r, b_r, W0_r, W1_r) = refs[:11]
        out_r = refs[-1]
        cm0 = jnp.maximum(c0_r[...], 1.0)
        cm1 = jnp.maximum(c1_r[...], 1.0)
        a0 = jnp.concatenate([p00_r[...], p10_r[...]], axis=1)
        a1 = jnp.concatenate([p01_r[...], p11_r[...]], axis=1)
        y = (jnp.dot(x_r[...], Wroot_r[...], preferred_element_type=_f32) + b_r[...]
             + jnp.dot(a0, W0_r[...], preferred_element_type=_f32,
                       precision=lax.Precision.HIGHEST) / cm0
             + jnp.dot(a1, W1_r[...], preferred_element_type=_f32,
                       precision=lax.Precision.HIGHEST) / cm1)
        if head:
            (Wo1_r, bo1_r, Wo2_r, bo2_r) = refs[11:15]
            z = _leaky(jnp.dot(y, Wo1_r[...], preferred_element_type=_f32) + bo1_r[...])
            y = jnp.dot(z, Wo2_r[...], preferred_element_type=_f32) + bo2_r[...]
        out_r[...] = y

    in_specs = [
        pl.BlockSpec((_TB, EMB), lambda i: (i, 0)),
        pl.BlockSpec((_TB, HALF), lambda i: (i, 0)),
        pl.BlockSpec((_TB, HALF), lambda i: (i, 0)),
        pl.BlockSpec((_TB, HALF), lambda i: (i, 0)),
        pl.BlockSpec((_TB, HALF), lambda i: (i, 0)),
        pl.BlockSpec((_TB, 1), lambda i: (i, 0)),
        pl.BlockSpec((_TB, 1), lambda i: (i, 0)),
        pl.BlockSpec((EMB, EMB), lambda i: (0, 0)),
        pl.BlockSpec((1, EMB), lambda i: (0, 0)),
        pl.BlockSpec((EMB, EMB), lambda i: (0, 0)),
        pl.BlockSpec((EMB, EMB), lambda i: (0, 0)),
    ]
    args = [x, p00, p10, p01, p11, c0, c1, Wroot, b.reshape(1, EMB), W0, W1]
    if head:
        Wo1, bo1, Wo2p, bo2p = head
        in_specs += [
            pl.BlockSpec((EMB, EMB), lambda i: (0, 0)),
            pl.BlockSpec((1, EMB), lambda i: (0, 0)),
            pl.BlockSpec((EMB, EMB), lambda i: (0, 0)),
            pl.BlockSpec((1, EMB), lambda i: (0, 0)),
        ]
        args += [Wo1, bo1.reshape(1, EMB), Wo2p, bo2p]

    return pl.pallas_call(
        body,
        grid=(N // _TB,),
        in_specs=in_specs,
        out_specs=pl.BlockSpec((_TB, EMB), lambda i: (i, 0)),
        out_shape=jax.ShapeDtypeStruct((N, EMB), _f32),
    )(*args)


def kernel(des, tweet, num_prop, cat_prop, edge_index, edge_type, Wd, bd, Wt, bt,
           Wi, bi, Wrel, Wroot, brgcn, Wo1, bo1, Wo2, bo2):
    x = _tc_input(des, tweet, Wd, bd, Wt, bt, Wi, bi)

    src = edge_index[0]
    sidx = edge_index[1] + N * edge_type
    pad = EP - E
    gsrc = jnp.pad(src, (0, pad))
    sidx = jnp.pad(sidx, (0, pad), constant_values=2 * N)
    eidx = jnp.stack([gsrc.reshape(-1, CHUNK), sidx.reshape(-1, CHUNK)], axis=1)

    z2d = jnp.zeros((ROWS_PER_TILE, HALF), _f32)
    zc = jnp.zeros((ROWS_PER_TILE,), _f32)
    ones = jnp.ones((CHUNK,), _f32)

    sc_cnt = _make_sc_agg(True)
    sc_plain = _make_sc_agg(False)

    def split_halves(v):
        v = v.astype(jnp.bfloat16).astype(_f32)
        v = jnp.pad(v, ((0, NPAD - N), (0, 0)))
        return jnp.concatenate([v[:, :HALF], v[:, HALF:]], axis=0)

    def pieces(agg):
        p = lambda c_, r_: lax.dynamic_slice_in_dim(agg, c_ * ACC + r_ * N, N)
        return p(0, 0), p(1, 0), p(0, 1), p(1, 1)

    W0 = Wrel[0].astype(jnp.bfloat16).astype(_f32)
    W1 = Wrel[1].astype(jnp.bfloat16).astype(_f32)
    Wo2p = jnp.pad(Wo2, ((0, 0), (0, EMB - 2)))
    bo2p = jnp.pad(bo2, (0, EMB - 2)).reshape(1, EMB)

    agg1, cnt = sc_cnt(split_halves(x), eidx, z2d, zc, ones)
    c0 = cnt[:N].reshape(N, 1)
    c1 = cnt[N:2 * N].reshape(N, 1)
    p00, p10, p01, p11 = pieces(agg1)
    x1 = _tc_combine(x, p00, p10, p01, p11, c0, c1, Wroot, brgcn, W0, W1, None)

    (agg2,) = sc_plain(split_halves(x1), eidx, z2d, zc, ones)
    q00, q10, q01, q11 = pieces(agg2)
    out = _tc_combine(x1, q00, q10, q01, q11, c0, c1, Wroot, brgcn, W0, W1,
                      (Wo1, bo1, Wo2p, bo2p))
    return out[:, :2]


# speedup vs baseline: 10.3901x; 1.0456x over previous
import jax
import jax.numpy as jnp
from jax import lax
from jax.experimental import pallas as pl
from jax.experimental.pallas import tpu as pltpu
from jax.experimental.pallas import tpu_sc as plsc

N = 10000
E = 320000
EMB = 128
HALF = 64
NUM_REL = 2

NTILE = 16
CHUNK = 32
NBUF = 3
ACC = 20096
ROWS_PER_TILE = ACC // NTILE
EP = 321024
PER_TILE_E = EP // NTILE
NCHUNK = PER_TILE_E // CHUNK
NGRP = NCHUNK // NBUF
TROWS = N // NTILE

_f32 = jnp.float32


def _leaky(x):
    return jnp.where(x >= 0, x, 0.01 * x)


def _make_sc_agg(with_cnt: bool):
    mesh = plsc.VectorSubcoreMesh(
        core_axis_name="c", subcore_axis_name="s", num_cores=2, num_subcores=NTILE
    )
    out_type = [jax.ShapeDtypeStruct((2 * ACC, HALF), _f32)]
    if with_cnt:
        out_type.append(jax.ShapeDtypeStruct((ACC,), _f32))

    def body(*refs):
        (xh0, xh1, eidx, z2d, zc, ones_h) = refs[:6]
        nout = 2 if with_cnt else 1
        outs = refs[6:6 + nout]
        agg = outs[0]
        cnt = outs[1] if with_cnt else None
        rest = list(refs[6 + nout:])
        eidx_v = rest[0:NBUF]
        gidx_v = rest[NBUF:2 * NBUF]
        sidx_v = rest[2 * NBUF:3 * NBUF]
        rows_v = rest[3 * NBUF:4 * NBUF]
        (ones_v, acc, acc_cnt, xtab, sem_i, sem_g, sem_s, sem_c) = rest[4 * NBUF:]

        c = lax.axis_index("c")
        s = lax.axis_index("s")
        zbase = s * ROWS_PER_TILE
        tbase = s * TROWS

        @pl.when(c == 0)
        def _():
            pltpu.async_copy(xh0.at[pl.ds(tbase, TROWS)],
                             xtab.at[pl.ds(tbase, TROWS)], sem_i)

        @pl.when(c == 1)
        def _():
            pltpu.async_copy(xh1.at[pl.ds(tbase, TROWS)],
                             xtab.at[pl.ds(tbase, TROWS)], sem_i)
        pltpu.async_copy(z2d, acc.at[pl.ds(zbase, ROWS_PER_TILE)], sem_i)
        if with_cnt:
            @pl.when(c == 0)
            def _():
                pltpu.async_copy(zc, acc_cnt.at[pl.ds(zbase, ROWS_PER_TILE)], sem_i)
                pltpu.sync_copy(ones_h, ones_v)
        @pl.when(c == 0)
        def _():
            pltpu.make_async_copy(xh0.at[pl.ds(tbase, TROWS)],
                                  xtab.at[pl.ds(tbase, TROWS)], sem_i).wait()

        @pl.when(c == 1)
        def _():
            pltpu.make_async_copy(xh1.at[pl.ds(tbase, TROWS)],
                                  xtab.at[pl.ds(tbase, TROWS)], sem_i).wait()
        pltpu.make_async_copy(z2d, acc.at[pl.ds(zbase, ROWS_PER_TILE)], sem_i).wait()
        if with_cnt:
            @pl.when(c == 0)
            def _():
                pltpu.make_async_copy(
                    zc, acc_cnt.at[pl.ds(zbase, ROWS_PER_TILE)], sem_i).wait()
        plsc.subcore_barrier()

        cbase = s * NCHUNK

        for b in range(NBUF):
            pltpu.async_copy(eidx.at[cbase + b], eidx_v[b], sem_i)

        def grp(g, carry):
            k0 = g * NBUF
            for b in range(NBUF):
                cid = cbase + k0 + b

                @pl.when(g > 0)
                def _(b=b, cid=cid):
                    pltpu.make_async_copy(rows_v[b], acc.at[sidx_v[b]], sem_s).wait()
                    if with_cnt:
                        @pl.when(c == 0)
                        def _():
                            pltpu.make_async_copy(
                                ones_v, acc_cnt.at[sidx_v[b]], sem_c).wait()

                pltpu.make_async_copy(eidx.at[cid], eidx_v[b], sem_i).wait()
                for j in range(CHUNK // 16):
                    sl = pl.ds(j * 16, 16)
                    gidx_v[b][sl] = eidx_v[b][0, sl]
                    sidx_v[b][sl] = eidx_v[b][1, sl]

                @pl.when(g < NGRP - 1)
                def _(b=b, cid=cid):
                    pltpu.async_copy(eidx.at[cid + NBUF], eidx_v[b], sem_i)

                pltpu.async_copy(xtab.at[gidx_v[b]], rows_v[b], sem_g)

            for b in range(NBUF):
                pltpu.make_async_copy(xtab.at[gidx_v[b]], rows_v[b], sem_g).wait()
                pltpu.async_copy(rows_v[b], acc.at[sidx_v[b]], sem_s, add=True)
                if with_cnt:
                    @pl.when(c == 0)
                    def _(b=b):
                        pltpu.async_copy(ones_v, acc_cnt.at[sidx_v[b]], sem_c, add=True)
            return carry

        lax.fori_loop(0, NGRP, grp, 0)

        for b in range(NBUF):
            pltpu.make_async_copy(rows_v[b], acc.at[sidx_v[b]], sem_s).wait()
            if with_cnt:
                @pl.when(c == 0)
                def _(b=b):
                    pltpu.make_async_copy(ones_v, acc_cnt.at[sidx_v[b]], sem_c).wait()
        plsc.subcore_barrier()

        pltpu.sync_copy(acc.at[pl.ds(zbase, ROWS_PER_TILE)],
                        agg.at[pl.ds(c * ACC + zbase, ROWS_PER_TILE)])
        if with_cnt:
            @pl.when(c == 0)
            def _():
                pltpu.sync_copy(acc_cnt.at[pl.ds(zbase, ROWS_PER_TILE)],
                                cnt.at[pl.ds(zbase, ROWS_PER_TILE)])

    return pl.kernel(
        body,
        out_type=out_type,
        mesh=mesh,
        scratch_types=(
            [pltpu.VMEM((2, CHUNK), jnp.int32) for _ in range(NBUF)]
            + [pltpu.VMEM((CHUNK,), jnp.int32) for _ in range(NBUF)]
            + [pltpu.VMEM((CHUNK,), jnp.int32) for _ in range(NBUF)]
            + [pltpu.VMEM((CHUNK, HALF), _f32) for _ in range(NBUF)]
            + [
                pltpu.VMEM((CHUNK,), _f32),
                pltpu.VMEM_SHARED((ACC, HALF), _f32),
                pltpu.VMEM_SHARED((ACC,), _f32),
                pltpu.VMEM_SHARED((N, HALF), _f32),
                pltpu.SemaphoreType.DMA,
                pltpu.SemaphoreType.DMA,
                pltpu.SemaphoreType.DMA,
                pltpu.SemaphoreType.DMA,
            ]
        ),
        compiler_params=pltpu.CompilerParams(use_tc_tiling_on_sc=False),
    )


_TB = 1000


def _tc_input(des, tweet, Wd, bd, Wt, bt, Wi, bi):
    def body(des_r, tw_r, Wd_r, bd_r, Wt_r, bt_r, Wi_r, bi_r, out_r, h0_r, h1_r):
        d = _leaky(jnp.dot(des_r[...], Wd_r[...], preferred_element_type=_f32) + bd_r[...])
        t = _leaky(jnp.dot(tw_r[...], Wt_r[...], preferred_element_type=_f32) + bt_r[...])
        xc = jnp.concatenate([d, t], axis=1)
        xv = _leaky(jnp.dot(xc, Wi_r[...], preferred_element_type=_f32) + bi_r[...])
        out_r[...] = xv
        hv = xv.astype(jnp.bfloat16).astype(_f32)
        h0_r[...] = hv[:, :HALF]
        h1_r[...] = hv[:, HALF:]

    return pl.pallas_call(
        body,
        grid=(N // _TB,),
        in_specs=[
            pl.BlockSpec((_TB, 768), lambda i: (i, 0)),
            pl.BlockSpec((_TB, 768), lambda i: (i, 0)),
            pl.BlockSpec((768, HALF), lambda i: (0, 0)),
            pl.BlockSpec((1, HALF), lambda i: (0, 0)),
            pl.BlockSpec((768, HALF), lambda i: (0, 0)),
            pl.BlockSpec((1, HALF), lambda i: (0, 0)),
            pl.BlockSpec((EMB, EMB), lambda i: (0, 0)),
            pl.BlockSpec((1, EMB), lambda i: (0, 0)),
        ],
        out_specs=[
            pl.BlockSpec((_TB, EMB), lambda i: (i, 0)),
            pl.BlockSpec((_TB, HALF), lambda i: (i, 0)),
            pl.BlockSpec((_TB, HALF), lambda i: (i, 0)),
        ],
        out_shape=[
            jax.ShapeDtypeStruct((N, EMB), _f32),
            jax.ShapeDtypeStruct((N, HALF), _f32),
            jax.ShapeDtypeStruct((N, HALF), _f32),
        ],
    )(des, tweet, Wd, bd.reshape(1, HALF), Wt, bt.reshape(1, HALF), Wi, bi.reshape(1, EMB))


def _tc_combine(x, p00, p10, p01, p11, c0, c1, Wroot, b, W0, W1, head):
    def body(*refs):
        (x_r, p00_r, p10_r, p01_r, p11_r, c0_r, c1_r, Wroot_r, b_r, W0_r, W1_r) = refs[:11]
        if head:
            out_r = refs[-1]
        cm0 = jnp.maximum(c0_r[...], 1.0)
        cm1 = jnp.maximum(c1_r[...], 1.0)
        a0 = jnp.concatenate([p00_r[...], p10_r[...]], axis=1)
        a1 = jnp.concatenate([p01_r[...], p11_r[...]], axis=1)
        y = (jnp.dot(x_r[...], Wroot_r[...], preferred_element_type=_f32) + b_r[...]
             + jnp.dot(a0, W0_r[...], preferred_element_type=_f32,
                       precision=lax.Precision.HIGHEST) / cm0
             + jnp.dot(a1, W1_r[...], preferred_element_type=_f32,
                       precision=lax.Precision.HIGHEST) / cm1)
        if head:
            (Wo1_r, bo1_r, Wo2_r, bo2_r) = refs[11:15]
            z = _leaky(jnp.dot(y, Wo1_r[...], preferred_element_type=_f32) + bo1_r[...])
            y = jnp.dot(z, Wo2_r[...], preferred_element_type=_f32) + bo2_r[...]
            out_r[...] = y
        else:
            out_r, h0_r, h1_r = refs[-3:]
            out_r[...] = y
            hv = y.astype(jnp.bfloat16).astype(_f32)
            h0_r[...] = hv[:, :HALF]
            h1_r[...] = hv[:, HALF:]

    in_specs = [
        pl.BlockSpec((_TB, EMB), lambda i: (i, 0)),
        pl.BlockSpec((_TB, HALF), lambda i: (i, 0)),
        pl.BlockSpec((_TB, HALF), lambda i: (i, 0)),
        pl.BlockSpec((_TB, HALF), lambda i: (i, 0)),
        pl.BlockSpec((_TB, HALF), lambda i: (i, 0)),
        pl.BlockSpec((_TB, 1), lambda i: (i, 0)),
        pl.BlockSpec((_TB, 1), lambda i: (i, 0)),
        pl.BlockSpec((EMB, EMB), lambda i: (0, 0)),
        pl.BlockSpec((1, EMB), lambda i: (0, 0)),
        pl.BlockSpec((EMB, EMB), lambda i: (0, 0)),
        pl.BlockSpec((EMB, EMB), lambda i: (0, 0)),
    ]
    args = [x, p00, p10, p01, p11, c0, c1, Wroot, b.reshape(1, EMB), W0, W1]
    if head:
        Wo1, bo1, Wo2p, bo2p = head
        in_specs += [
            pl.BlockSpec((EMB, EMB), lambda i: (0, 0)),
            pl.BlockSpec((1, EMB), lambda i: (0, 0)),
            pl.BlockSpec((EMB, EMB), lambda i: (0, 0)),
            pl.BlockSpec((1, EMB), lambda i: (0, 0)),
        ]
        args += [Wo1, bo1.reshape(1, EMB), Wo2p, bo2p]

    if head:
        out_specs = pl.BlockSpec((_TB, EMB), lambda i: (i, 0))
        out_shape = jax.ShapeDtypeStruct((N, EMB), _f32)
    else:
        out_specs = [
            pl.BlockSpec((_TB, EMB), lambda i: (i, 0)),
            pl.BlockSpec((_TB, HALF), lambda i: (i, 0)),
            pl.BlockSpec((_TB, HALF), lambda i: (i, 0)),
        ]
        out_shape = [
            jax.ShapeDtypeStruct((N, EMB), _f32),
            jax.ShapeDtypeStruct((N, HALF), _f32),
            jax.ShapeDtypeStruct((N, HALF), _f32),
        ]
    return pl.pallas_call(
        body,
        grid=(N // _TB,),
        in_specs=in_specs,
        out_specs=out_specs,
        out_shape=out_shape,
    )(*args)


def kernel(des, tweet, num_prop, cat_prop, edge_index, edge_type, Wd, bd, Wt, bt,
           Wi, bi, Wrel, Wroot, brgcn, Wo1, bo1, Wo2, bo2):
    x, xh0, xh1 = _tc_input(des, tweet, Wd, bd, Wt, bt, Wi, bi)

    src = edge_index[0]
    sidx = edge_index[1] + N * edge_type
    pad = EP - E
    gsrc = jnp.pad(src, (0, pad))
    sidx = jnp.pad(sidx, (0, pad), constant_values=2 * N)
    eidx = jnp.stack([gsrc.reshape(-1, CHUNK), sidx.reshape(-1, CHUNK)], axis=1)

    z2d = jnp.zeros((ROWS_PER_TILE, HALF), _f32)
    zc = jnp.zeros((ROWS_PER_TILE,), _f32)
    ones = jnp.ones((CHUNK,), _f32)

    sc_cnt = _make_sc_agg(True)
    sc_plain = _make_sc_agg(False)

    def pieces(agg):
        p = lambda c_, r_: lax.dynamic_slice_in_dim(agg, c_ * ACC + r_ * N, N)
        return p(0, 0), p(1, 0), p(0, 1), p(1, 1)

    W0 = Wrel[0].astype(jnp.bfloat16).astype(_f32)
    W1 = Wrel[1].astype(jnp.bfloat16).astype(_f32)
    Wo2p = jnp.pad(Wo2, ((0, 0), (0, EMB - 2)))
    bo2p = jnp.pad(bo2, (0, EMB - 2)).reshape(1, EMB)

    agg1, cnt = sc_cnt(xh0, xh1, eidx, z2d, zc, ones)
    c0 = cnt[:N].reshape(N, 1)
    c1 = cnt[N:2 * N].reshape(N, 1)
    p00, p10, p01, p11 = pieces(agg1)
    x1, y0, y1 = _tc_combine(x, p00, p10, p01, p11, c0, c1, Wroot, brgcn, W0, W1, None)

    (agg2,) = sc_plain(y0, y1, eidx, z2d, zc, ones)
    q00, q10, q01, q11 = pieces(agg2)
    out = _tc_combine(x1, q00, q10, q01, q11, c0, c1, Wroot, brgcn, W0, W1,
                      (Wo1, bo1, Wo2p, bo2p))
    return out[:, :2]


# speedup vs baseline: 10.5028x; 1.0108x over previous
import jax
import jax.numpy as jnp
from jax import lax
from jax.experimental import pallas as pl
from jax.experimental.pallas import tpu as pltpu
from jax.experimental.pallas import tpu_sc as plsc

N = 10000
E = 320000
EMB = 128
HALF = 64
NUM_REL = 2

NTILE = 16
CHUNK = 32
NBUF = 4
ACC = 20096
ROWS_PER_TILE = ACC // NTILE
EP = 321536
PER_TILE_E = EP // NTILE
NCHUNK = PER_TILE_E // CHUNK
NGRP = NCHUNK // NBUF
TROWS = N // NTILE

_f32 = jnp.float32


def _leaky(x):
    return jnp.where(x >= 0, x, 0.01 * x)


def _make_sc_agg(with_cnt: bool):
    mesh = plsc.VectorSubcoreMesh(
        core_axis_name="c", subcore_axis_name="s", num_cores=2, num_subcores=NTILE
    )
    out_type = [jax.ShapeDtypeStruct((2 * ACC, HALF), _f32)]
    if with_cnt:
        out_type.append(jax.ShapeDtypeStruct((ACC,), _f32))

    def body(*refs):
        (xh0, xh1, eidx, z2d, zc, ones_h) = refs[:6]
        nout = 2 if with_cnt else 1
        outs = refs[6:6 + nout]
        agg = outs[0]
        cnt = outs[1] if with_cnt else None
        rest = list(refs[6 + nout:])
        eidx_v = rest[0:NBUF]
        gidx_v = rest[NBUF:2 * NBUF]
        sidx_v = rest[2 * NBUF:3 * NBUF]
        rows_v = rest[3 * NBUF:4 * NBUF]
        (ones_v, acc, acc_cnt, xtab, sem_i, sem_g, sem_s, sem_c) = rest[4 * NBUF:]

        c = lax.axis_index("c")
        s = lax.axis_index("s")
        zbase = s * ROWS_PER_TILE
        tbase = s * TROWS

        @pl.when(c == 0)
        def _():
            pltpu.async_copy(xh0.at[pl.ds(tbase, TROWS)],
                             xtab.at[pl.ds(tbase, TROWS)], sem_i)

        @pl.when(c == 1)
        def _():
            pltpu.async_copy(xh1.at[pl.ds(tbase, TROWS)],
                             xtab.at[pl.ds(tbase, TROWS)], sem_i)
        pltpu.async_copy(z2d, acc.at[pl.ds(zbase, ROWS_PER_TILE)], sem_i)
        if with_cnt:
            @pl.when(c == 0)
            def _():
                pltpu.async_copy(zc, acc_cnt.at[pl.ds(zbase, ROWS_PER_TILE)], sem_i)
                pltpu.sync_copy(ones_h, ones_v)
        @pl.when(c == 0)
        def _():
            pltpu.make_async_copy(xh0.at[pl.ds(tbase, TROWS)],
                                  xtab.at[pl.ds(tbase, TROWS)], sem_i).wait()

        @pl.when(c == 1)
        def _():
            pltpu.make_async_copy(xh1.at[pl.ds(tbase, TROWS)],
                                  xtab.at[pl.ds(tbase, TROWS)], sem_i).wait()
        pltpu.make_async_copy(z2d, acc.at[pl.ds(zbase, ROWS_PER_TILE)], sem_i).wait()
        if with_cnt:
            @pl.when(c == 0)
            def _():
                pltpu.make_async_copy(
                    zc, acc_cnt.at[pl.ds(zbase, ROWS_PER_TILE)], sem_i).wait()
        plsc.subcore_barrier()

        cbase = s * NCHUNK

        for b in range(NBUF):
            pltpu.async_copy(eidx.at[cbase + b], eidx_v[b], sem_i)

        def grp(g, carry):
            k0 = g * NBUF
            for b in range(NBUF):
                cid = cbase + k0 + b

                @pl.when(g > 0)
                def _(b=b, cid=cid):
                    pltpu.make_async_copy(rows_v[b], acc.at[sidx_v[b]], sem_s).wait()
                    if with_cnt:
                        @pl.when(c == 0)
                        def _():
                            pltpu.make_async_copy(
                                ones_v, acc_cnt.at[sidx_v[b]], sem_c).wait()

                pltpu.make_async_copy(eidx.at[cid], eidx_v[b], sem_i).wait()
                for j in range(CHUNK // 16):
                    sl = pl.ds(j * 16, 16)
                    gidx_v[b][sl] = eidx_v[b][0, sl]
                    sidx_v[b][sl] = eidx_v[b][1, sl]

                @pl.when(g < NGRP - 1)
                def _(b=b, cid=cid):
                    pltpu.async_copy(eidx.at[cid + NBUF], eidx_v[b], sem_i)

                pltpu.async_copy(xtab.at[gidx_v[b]], rows_v[b], sem_g)

            for b in range(NBUF):
                pltpu.make_async_copy(xtab.at[gidx_v[b]], rows_v[b], sem_g).wait()
                pltpu.async_copy(rows_v[b], acc.at[sidx_v[b]], sem_s, add=True)
                if with_cnt:
                    @pl.when(c == 0)
                    def _(b=b):
                        pltpu.async_copy(ones_v, acc_cnt.at[sidx_v[b]], sem_c, add=True)
            return carry

        lax.fori_loop(0, NGRP, grp, 0)

        for b in range(NBUF):
            pltpu.make_async_copy(rows_v[b], acc.at[sidx_v[b]], sem_s).wait()
            if with_cnt:
                @pl.when(c == 0)
                def _(b=b):
                    pltpu.make_async_copy(ones_v, acc_cnt.at[sidx_v[b]], sem_c).wait()
        plsc.subcore_barrier()

        pltpu.sync_copy(acc.at[pl.ds(zbase, ROWS_PER_TILE)],
                        agg.at[pl.ds(c * ACC + zbase, ROWS_PER_TILE)])
        if with_cnt:
            @pl.when(c == 0)
            def _():
                pltpu.sync_copy(acc_cnt.at[pl.ds(zbase, ROWS_PER_TILE)],
                                cnt.at[pl.ds(zbase, ROWS_PER_TILE)])

    return pl.kernel(
        body,
        out_type=out_type,
        mesh=mesh,
        scratch_types=(
            [pltpu.VMEM((2, CHUNK), jnp.int32) for _ in range(NBUF)]
            + [pltpu.VMEM((CHUNK,), jnp.int32) for _ in range(NBUF)]
            + [pltpu.VMEM((CHUNK,), jnp.int32) for _ in range(NBUF)]
            + [pltpu.VMEM((CHUNK, HALF), _f32) for _ in range(NBUF)]
            + [
                pltpu.VMEM((CHUNK,), _f32),
                pltpu.VMEM_SHARED((ACC, HALF), _f32),
                pltpu.VMEM_SHARED((ACC,), _f32),
                pltpu.VMEM_SHARED((N, HALF), _f32),
                pltpu.SemaphoreType.DMA,
                pltpu.SemaphoreType.DMA,
                pltpu.SemaphoreType.DMA,
                pltpu.SemaphoreType.DMA,
            ]
        ),
        compiler_params=pltpu.CompilerParams(use_tc_tiling_on_sc=False),
    )


_TB = 1000


def _tc_input(des, tweet, Wd, bd, Wt, bt, Wi, bi):
    def body(des_r, tw_r, Wd_r, bd_r, Wt_r, bt_r, Wi_r, bi_r, out_r, h0_r, h1_r):
        d = _leaky(jnp.dot(des_r[...], Wd_r[...], preferred_element_type=_f32) + bd_r[...])
        t = _leaky(jnp.dot(tw_r[...], Wt_r[...], preferred_element_type=_f32) + bt_r[...])
        xc = jnp.concatenate([d, t], axis=1)
        xv = _leaky(jnp.dot(xc, Wi_r[...], preferred_element_type=_f32) + bi_r[...])
        out_r[...] = xv
        hv = xv.astype(jnp.bfloat16).astype(_f32)
        h0_r[...] = hv[:, :HALF]
        h1_r[...] = hv[:, HALF:]

    return pl.pallas_call(
        body,
        grid=(N // _TB,),
        in_specs=[
            pl.BlockSpec((_TB, 768), lambda i: (i, 0)),
            pl.BlockSpec((_TB, 768), lambda i: (i, 0)),
            pl.BlockSpec((768, HALF), lambda i: (0, 0)),
            pl.BlockSpec((1, HALF), lambda i: (0, 0)),
            pl.BlockSpec((768, HALF), lambda i: (0, 0)),
            pl.BlockSpec((1, HALF), lambda i: (0, 0)),
            pl.BlockSpec((EMB, EMB), lambda i: (0, 0)),
            pl.BlockSpec((1, EMB), lambda i: (0, 0)),
        ],
        out_specs=[
            pl.BlockSpec((_TB, EMB), lambda i: (i, 0)),
            pl.BlockSpec((_TB, HALF), lambda i: (i, 0)),
            pl.BlockSpec((_TB, HALF), lambda i: (i, 0)),
        ],
        out_shape=[
            jax.ShapeDtypeStruct((N, EMB), _f32),
            jax.ShapeDtypeStruct((N, HALF), _f32),
            jax.ShapeDtypeStruct((N, HALF), _f32),
        ],
    )(des, tweet, Wd, bd.reshape(1, HALF), Wt, bt.reshape(1, HALF), Wi, bi.reshape(1, EMB))


def _tc_combine(x, p00, p10, p01, p11, c0, c1, Wroot, b, W0, W1, head):
    def body(*refs):
        (x_r, p00_r, p10_r, p01_r, p11_r, c0_r, c1_r, Wroot_r, b_r, W0_r, W1_r) = refs[:11]
        if head:
            out_r = refs[-1]
        cm0 = jnp.maximum(c0_r[...], 1.0)
        cm1 = jnp.maximum(c1_r[...], 1.0)
        a0 = jnp.concatenate([p00_r[...], p10_r[...]], axis=1)
        a1 = jnp.concatenate([p01_r[...], p11_r[...]], axis=1)
        y = (jnp.dot(x_r[...], Wroot_r[...], preferred_element_type=_f32) + b_r[...]
             + jnp.dot(a0, W0_r[...], preferred_element_type=_f32,
                       precision=lax.Precision.HIGHEST) / cm0
             + jnp.dot(a1, W1_r[...], preferred_element_type=_f32,
                       precision=lax.Precision.HIGHEST) / cm1)
        if head:
            (Wo1_r, bo1_r, Wo2_r, bo2_r) = refs[11:15]
            z = _leaky(jnp.dot(y, Wo1_r[...], preferred_element_type=_f32) + bo1_r[...])
            y = jnp.dot(z, Wo2_r[...], preferred_element_type=_f32) + bo2_r[...]
            out_r[...] = y
        else:
            out_r, h0_r, h1_r = refs[-3:]
            out_r[...] = y
            hv = y.astype(jnp.bfloat16).astype(_f32)
            h0_r[...] = hv[:, :HALF]
            h1_r[...] = hv[:, HALF:]

    in_specs = [
        pl.BlockSpec((_TB, EMB), lambda i: (i, 0)),
        pl.BlockSpec((_TB, HALF), lambda i: (i, 0)),
        pl.BlockSpec((_TB, HALF), lambda i: (i, 0)),
        pl.BlockSpec((_TB, HALF), lambda i: (i, 0)),
        pl.BlockSpec((_TB, HALF), lambda i: (i, 0)),
        pl.BlockSpec((_TB, 1), lambda i: (i, 0)),
        pl.BlockSpec((_TB, 1), lambda i: (i, 0)),
        pl.BlockSpec((EMB, EMB), lambda i: (0, 0)),
        pl.BlockSpec((1, EMB), lambda i: (0, 0)),
        pl.BlockSpec((EMB, EMB), lambda i: (0, 0)),
        pl.BlockSpec((EMB, EMB), lambda i: (0, 0)),
    ]
    args = [x, p00, p10, p01, p11, c0, c1, Wroot, b.reshape(1, EMB), W0, W1]
    if head:
        Wo1, bo1, Wo2p, bo2p = head
        in_specs += [
            pl.BlockSpec((EMB, EMB), lambda i: (0, 0)),
            pl.BlockSpec((1, EMB), lambda i: (0, 0)),
            pl.BlockSpec((EMB, EMB), lambda i: (0, 0)),
            pl.BlockSpec((1, EMB), lambda i: (0, 0)),
        ]
        args += [Wo1, bo1.reshape(1, EMB), Wo2p, bo2p]

    if head:
        out_specs = pl.BlockSpec((_TB, EMB), lambda i: (i, 0))
        out_shape = jax.ShapeDtypeStruct((N, EMB), _f32)
    else:
        out_specs = [
            pl.BlockSpec((_TB, EMB), lambda i: (i, 0)),
            pl.BlockSpec((_TB, HALF), lambda i: (i, 0)),
            pl.BlockSpec((_TB, HALF), lambda i: (i, 0)),
        ]
        out_shape = [
            jax.ShapeDtypeStruct((N, EMB), _f32),
            jax.ShapeDtypeStruct((N, HALF), _f32),
            jax.ShapeDtypeStruct((N, HALF), _f32),
        ]
    return pl.pallas_call(
        body,
        grid=(N // _TB,),
        in_specs=in_specs,
        out_specs=out_specs,
        out_shape=out_shape,
    )(*args)


def kernel(des, tweet, num_prop, cat_prop, edge_index, edge_type, Wd, bd, Wt, bt,
           Wi, bi, Wrel, Wroot, brgcn, Wo1, bo1, Wo2, bo2):
    x, xh0, xh1 = _tc_input(des, tweet, Wd, bd, Wt, bt, Wi, bi)

    src = edge_index[0]
    sidx = edge_index[1] + N * edge_type
    pad = EP - E
    gsrc = jnp.pad(src, (0, pad))
    sidx = jnp.pad(sidx, (0, pad), constant_values=2 * N)
    eidx = jnp.stack([gsrc.reshape(-1, CHUNK), sidx.reshape(-1, CHUNK)], axis=1)

    z2d = jnp.zeros((ROWS_PER_TILE, HALF), _f32)
    zc = jnp.zeros((ROWS_PER_TILE,), _f32)
    ones = jnp.ones((CHUNK,), _f32)

    sc_cnt = _make_sc_agg(True)
    sc_plain = _make_sc_agg(False)

    def pieces(agg):
        p = lambda c_, r_: lax.dynamic_slice_in_dim(agg, c_ * ACC + r_ * N, N)
        return p(0, 0), p(1, 0), p(0, 1), p(1, 1)

    W0 = Wrel[0].astype(jnp.bfloat16).astype(_f32)
    W1 = Wrel[1].astype(jnp.bfloat16).astype(_f32)
    Wo2p = jnp.pad(Wo2, ((0, 0), (0, EMB - 2)))
    bo2p = jnp.pad(bo2, (0, EMB - 2)).reshape(1, EMB)

    agg1, cnt = sc_cnt(xh0, xh1, eidx, z2d, zc, ones)
    c0 = cnt[:N].reshape(N, 1)
    c1 = cnt[N:2 * N].reshape(N, 1)
    p00, p10, p01, p11 = pieces(agg1)
    x1, y0, y1 = _tc_combine(x, p00, p10, p01, p11, c0, c1, Wroot, brgcn, W0, W1, None)

    (agg2,) = sc_plain(y0, y1, eidx, z2d, zc, ones)
    q00, q10, q01, q11 = pieces(agg2)
    out = _tc_combine(x1, q00, q10, q01, q11, c0, c1, Wroot, brgcn, W0, W1,
                      (Wo1, bo1, Wo2p, bo2p))
    return out[:, :2]


# speedup vs baseline: 10.5761x; 1.0070x over previous
import jax
import jax.numpy as jnp
from jax import lax
from jax.experimental import pallas as pl
from jax.experimental.pallas import tpu as pltpu
from jax.experimental.pallas import tpu_sc as plsc

N = 10000
E = 320000
EMB = 128
HALF = 64
NUM_REL = 2

NTILE = 16
CHUNK = 64
NBUF = 2
ACC = 20096
ROWS_PER_TILE = ACC // NTILE
EP = 321536
PER_TILE_E = EP // NTILE
NCHUNK = PER_TILE_E // CHUNK
NGRP = NCHUNK // NBUF
TROWS = N // NTILE

_f32 = jnp.float32


def _leaky(x):
    return jnp.where(x >= 0, x, 0.01 * x)


def _make_sc_agg(with_cnt: bool):
    mesh = plsc.VectorSubcoreMesh(
        core_axis_name="c", subcore_axis_name="s", num_cores=2, num_subcores=NTILE
    )
    out_type = [jax.ShapeDtypeStruct((2 * ACC, HALF), _f32)]
    if with_cnt:
        out_type.append(jax.ShapeDtypeStruct((ACC,), _f32))

    def body(*refs):
        (xh0, xh1, eidx, z2d, zc, ones_h) = refs[:6]
        nout = 2 if with_cnt else 1
        outs = refs[6:6 + nout]
        agg = outs[0]
        cnt = outs[1] if with_cnt else None
        rest = list(refs[6 + nout:])
        eidx_v = rest[0:NBUF]
        gidx_v = rest[NBUF:2 * NBUF]
        sidx_v = rest[2 * NBUF:3 * NBUF]
        rows_v = rest[3 * NBUF:4 * NBUF]
        (ones_v, acc, acc_cnt, xtab, sem_i, sem_g, sem_s, sem_c) = rest[4 * NBUF:]

        c = lax.axis_index("c")
        s = lax.axis_index("s")
        zbase = s * ROWS_PER_TILE
        tbase = s * TROWS

        @pl.when(c == 0)
        def _():
            pltpu.async_copy(xh0.at[pl.ds(tbase, TROWS)],
                             xtab.at[pl.ds(tbase, TROWS)], sem_i)

        @pl.when(c == 1)
        def _():
            pltpu.async_copy(xh1.at[pl.ds(tbase, TROWS)],
                             xtab.at[pl.ds(tbase, TROWS)], sem_i)
        pltpu.async_copy(z2d, acc.at[pl.ds(zbase, ROWS_PER_TILE)], sem_i)
        if with_cnt:
            @pl.when(c == 0)
            def _():
                pltpu.async_copy(zc, acc_cnt.at[pl.ds(zbase, ROWS_PER_TILE)], sem_i)
                pltpu.sync_copy(ones_h, ones_v)
        @pl.when(c == 0)
        def _():
            pltpu.make_async_copy(xh0.at[pl.ds(tbase, TROWS)],
                                  xtab.at[pl.ds(tbase, TROWS)], sem_i).wait()

        @pl.when(c == 1)
        def _():
            pltpu.make_async_copy(xh1.at[pl.ds(tbase, TROWS)],
                                  xtab.at[pl.ds(tbase, TROWS)], sem_i).wait()
        pltpu.make_async_copy(z2d, acc.at[pl.ds(zbase, ROWS_PER_TILE)], sem_i).wait()
        if with_cnt:
            @pl.when(c == 0)
            def _():
                pltpu.make_async_copy(
                    zc, acc_cnt.at[pl.ds(zbase, ROWS_PER_TILE)], sem_i).wait()
        plsc.subcore_barrier()

        cbase = s * NCHUNK

        for b in range(NBUF):
            pltpu.async_copy(eidx.at[cbase + b], eidx_v[b], sem_i)

        def grp(g, carry):
            k0 = g * NBUF
            for b in range(NBUF):
                cid = cbase + k0 + b

                @pl.when(g > 0)
                def _(b=b, cid=cid):
                    pltpu.make_async_copy(rows_v[b], acc.at[sidx_v[b]], sem_s).wait()
                    if with_cnt:
                        @pl.when(c == 0)
                        def _():
                            pltpu.make_async_copy(
                                ones_v, acc_cnt.at[sidx_v[b]], sem_c).wait()

                pltpu.make_async_copy(eidx.at[cid], eidx_v[b], sem_i).wait()
                for j in range(CHUNK // 16):
                    sl = pl.ds(j * 16, 16)
                    gidx_v[b][sl] = eidx_v[b][0, sl]
                    sidx_v[b][sl] = eidx_v[b][1, sl]

                @pl.when(g < NGRP - 1)
                def _(b=b, cid=cid):
                    pltpu.async_copy(eidx.at[cid + NBUF], eidx_v[b], sem_i)

                pltpu.async_copy(xtab.at[gidx_v[b]], rows_v[b], sem_g)

            for b in range(NBUF):
                pltpu.make_async_copy(xtab.at[gidx_v[b]], rows_v[b], sem_g).wait()
                pltpu.async_copy(rows_v[b], acc.at[sidx_v[b]], sem_s, add=True)
                if with_cnt:
                    @pl.when(c == 0)
                    def _(b=b):
                        pltpu.async_copy(ones_v, acc_cnt.at[sidx_v[b]], sem_c, add=True)
            return carry

        lax.fori_loop(0, NGRP, grp, 0)

        for b in range(NBUF):
            pltpu.make_async_copy(rows_v[b], acc.at[sidx_v[b]], sem_s).wait()
            if with_cnt:
                @pl.when(c == 0)
                def _(b=b):
                    pltpu.make_async_copy(ones_v, acc_cnt.at[sidx_v[b]], sem_c).wait()
        plsc.subcore_barrier()

        pltpu.sync_copy(acc.at[pl.ds(zbase, ROWS_PER_TILE)],
                        agg.at[pl.ds(c * ACC + zbase, ROWS_PER_TILE)])
        if with_cnt:
            @pl.when(c == 0)
            def _():
                pltpu.sync_copy(acc_cnt.at[pl.ds(zbase, ROWS_PER_TILE)],
                                cnt.at[pl.ds(zbase, ROWS_PER_TILE)])

    return pl.kernel(
        body,
        out_type=out_type,
        mesh=mesh,
        scratch_types=(
            [pltpu.VMEM((2, CHUNK), jnp.int32) for _ in range(NBUF)]
            + [pltpu.VMEM((CHUNK,), jnp.int32) for _ in range(NBUF)]
            + [pltpu.VMEM((CHUNK,), jnp.int32) for _ in range(NBUF)]
            + [pltpu.VMEM((CHUNK, HALF), _f32) for _ in range(NBUF)]
            + [
                pltpu.VMEM((CHUNK,), _f32),
                pltpu.VMEM_SHARED((ACC, HALF), _f32),
                pltpu.VMEM_SHARED((ACC,), _f32),
                pltpu.VMEM_SHARED((N, HALF), _f32),
                pltpu.SemaphoreType.DMA,
                pltpu.SemaphoreType.DMA,
                pltpu.SemaphoreType.DMA,
                pltpu.SemaphoreType.DMA,
            ]
        ),
        compiler_params=pltpu.CompilerParams(use_tc_tiling_on_sc=False),
    )


_TB = 1000


def _tc_input(des, tweet, Wd, bd, Wt, bt, Wi, bi):
    def body(des_r, tw_r, Wd_r, bd_r, Wt_r, bt_r, Wi_r, bi_r, out_r, h0_r, h1_r):
        d = _leaky(jnp.dot(des_r[...], Wd_r[...], preferred_element_type=_f32) + bd_r[...])
        t = _leaky(jnp.dot(tw_r[...], Wt_r[...], preferred_element_type=_f32) + bt_r[...])
        xc = jnp.concatenate([d, t], axis=1)
        xv = _leaky(jnp.dot(xc, Wi_r[...], preferred_element_type=_f32) + bi_r[...])
        out_r[...] = xv
        hv = xv.astype(jnp.bfloat16).astype(_f32)
        h0_r[...] = hv[:, :HALF]
        h1_r[...] = hv[:, HALF:]

    return pl.pallas_call(
        body,
        grid=(N // _TB,),
        in_specs=[
            pl.BlockSpec((_TB, 768), lambda i: (i, 0)),
            pl.BlockSpec((_TB, 768), lambda i: (i, 0)),
            pl.BlockSpec((768, HALF), lambda i: (0, 0)),
            pl.BlockSpec((1, HALF), lambda i: (0, 0)),
            pl.BlockSpec((768, HALF), lambda i: (0, 0)),
            pl.BlockSpec((1, HALF), lambda i: (0, 0)),
            pl.BlockSpec((EMB, EMB), lambda i: (0, 0)),
            pl.BlockSpec((1, EMB), lambda i: (0, 0)),
        ],
        out_specs=[
            pl.BlockSpec((_TB, EMB), lambda i: (i, 0)),
            pl.BlockSpec((_TB, HALF), lambda i: (i, 0)),
            pl.BlockSpec((_TB, HALF), lambda i: (i, 0)),
        ],
        out_shape=[
            jax.ShapeDtypeStruct((N, EMB), _f32),
            jax.ShapeDtypeStruct((N, HALF), _f32),
            jax.ShapeDtypeStruct((N, HALF), _f32),
        ],
    )(des, tweet, Wd, bd.reshape(1, HALF), Wt, bt.reshape(1, HALF), Wi, bi.reshape(1, EMB))


def _tc_combine(x, p00, p10, p01, p11, c0, c1, Wroot, b, W0, W1, head):
    def body(*refs):
        (x_r, p00_r, p10_r, p01_r, p11_r, c0_r, c1_r, Wroot_r, b_r, W0_r, W1_r) = refs[:11]
        if head:
            out_r = refs[-1]
        cm0 = jnp.maximum(c0_r[...], 1.0)
        cm1 = jnp.maximum(c1_r[...], 1.0)
        a0 = jnp.concatenate([p00_r[...], p10_r[...]], axis=1)
        a1 = jnp.concatenate([p01_r[...], p11_r[...]], axis=1)
        y = (jnp.dot(x_r[...], Wroot_r[...], preferred_element_type=_f32) + b_r[...]
             + jnp.dot(a0, W0_r[...], preferred_element_type=_f32,
                       precision=lax.Precision.HIGHEST) / cm0
             + jnp.dot(a1, W1_r[...], preferred_element_type=_f32,
                       precision=lax.Precision.HIGHEST) / cm1)
        if head:
            (Wo1_r, bo1_r, Wo2_r, bo2_r) = refs[11:15]
            z = _leaky(jnp.dot(y, Wo1_r[...], preferred_element_type=_f32) + bo1_r[...])
            y = jnp.dot(z, Wo2_r[...], preferred_element_type=_f32) + bo2_r[...]
            out_r[...] = y
        else:
            out_r, h0_r, h1_r = refs[-3:]
            out_r[...] = y
            hv = y.astype(jnp.bfloat16).astype(_f32)
            h0_r[...] = hv[:, :HALF]
            h1_r[...] = hv[:, HALF:]

    in_specs = [
        pl.BlockSpec((_TB, EMB), lambda i: (i, 0)),
        pl.BlockSpec((_TB, HALF), lambda i: (i, 0)),
        pl.BlockSpec((_TB, HALF), lambda i: (i, 0)),
        pl.BlockSpec((_TB, HALF), lambda i: (i, 0)),
        pl.BlockSpec((_TB, HALF), lambda i: (i, 0)),
        pl.BlockSpec((_TB, 1), lambda i: (i, 0)),
        pl.BlockSpec((_TB, 1), lambda i: (i, 0)),
        pl.BlockSpec((EMB, EMB), lambda i: (0, 0)),
        pl.BlockSpec((1, EMB), lambda i: (0, 0)),
        pl.BlockSpec((EMB, EMB), lambda i: (0, 0)),
        pl.BlockSpec((EMB, EMB), lambda i: (0, 0)),
    ]
    args = [x, p00, p10, p01, p11, c0, c1, Wroot, b.reshape(1, EMB), W0, W1]
    if head:
        Wo1, bo1, Wo2p, bo2p = head
        in_specs += [
            pl.BlockSpec((EMB, EMB), lambda i: (0, 0)),
            pl.BlockSpec((1, EMB), lambda i: (0, 0)),
            pl.BlockSpec((EMB, EMB), lambda i: (0, 0)),
            pl.BlockSpec((1, EMB), lambda i: (0, 0)),
        ]
        args += [Wo1, bo1.reshape(1, EMB), Wo2p, bo2p]

    if head:
        out_specs = pl.BlockSpec((_TB, EMB), lambda i: (i, 0))
        out_shape = jax.ShapeDtypeStruct((N, EMB), _f32)
    else:
        out_specs = [
            pl.BlockSpec((_TB, EMB), lambda i: (i, 0)),
            pl.BlockSpec((_TB, HALF), lambda i: (i, 0)),
            pl.BlockSpec((_TB, HALF), lambda i: (i, 0)),
        ]
        out_shape = [
            jax.ShapeDtypeStruct((N, EMB), _f32),
            jax.ShapeDtypeStruct((N, HALF), _f32),
            jax.ShapeDtypeStruct((N, HALF), _f32),
        ]
    return pl.pallas_call(
        body,
        grid=(N // _TB,),
        in_specs=in_specs,
        out_specs=out_specs,
        out_shape=out_shape,
    )(*args)


def kernel(des, tweet, num_prop, cat_prop, edge_index, edge_type, Wd, bd, Wt, bt,
           Wi, bi, Wrel, Wroot, brgcn, Wo1, bo1, Wo2, bo2):
    x, xh0, xh1 = _tc_input(des, tweet, Wd, bd, Wt, bt, Wi, bi)

    src = edge_index[0]
    sidx = edge_index[1] + N * edge_type
    pad = EP - E
    gsrc = jnp.pad(src, (0, pad))
    sidx = jnp.pad(sidx, (0, pad), constant_values=2 * N)
    eidx = jnp.stack([gsrc.reshape(-1, CHUNK), sidx.reshape(-1, CHUNK)], axis=1)

    z2d = jnp.zeros((ROWS_PER_TILE, HALF), _f32)
    zc = jnp.zeros((ROWS_PER_TILE,), _f32)
    ones = jnp.ones((CHUNK,), _f32)

    sc_cnt = _make_sc_agg(True)
    sc_plain = _make_sc_agg(False)

    def pieces(agg):
        p = lambda c_, r_: lax.dynamic_slice_in_dim(agg, c_ * ACC + r_ * N, N)
        return p(0, 0), p(1, 0), p(0, 1), p(1, 1)

    W0 = Wrel[0].astype(jnp.bfloat16).astype(_f32)
    W1 = Wrel[1].astype(jnp.bfloat16).astype(_f32)
    Wo2p = jnp.pad(Wo2, ((0, 0), (0, EMB - 2)))
    bo2p = jnp.pad(bo2, (0, EMB - 2)).reshape(1, EMB)

    agg1, cnt = sc_cnt(xh0, xh1, eidx, z2d, zc, ones)
    c0 = cnt[:N].reshape(N, 1)
    c1 = cnt[N:2 * N].reshape(N, 1)
    p00, p10, p01, p11 = pieces(agg1)
    x1, y0, y1 = _tc_combine(x, p00, p10, p01, p11, c0, c1, Wroot, brgcn, W0, W1, None)

    (agg2,) = sc_plain(y0, y1, eidx, z2d, zc, ones)
    q00, q10, q01, q11 = pieces(agg2)
    out = _tc_combine(x1, q00, q10, q01, q11, c0, c1, Wroot, brgcn, W0, W1,
                      (Wo1, bo1, Wo2p, bo2p))
    return out[:, :2]
